# baseline (speedup 1.0000x reference)
import jax
import jax.numpy as jnp
from jax.experimental import pallas as pl
from jax.experimental.pallas import tpu as pltpu

E = 8
K = 2
H = 1024
FF = 2880
S = 2048
T = 128
PMAX = 5120
LIMIT = 7.0
HC = H // 4
FC = FF // 4
MAXTT = S // T


def _router_body(hid_ref, rw_ref, dest_ref, w_ref, cnt_ref, rank_ref):
    x = hid_ref[...]
    logits = jnp.dot(x, rw_ref[...], preferred_element_type=jnp.float32)
    eiota = jax.lax.broadcasted_iota(jnp.int32, (S, E), 1)
    m1 = jnp.max(logits, axis=1, keepdims=True)
    i1 = jnp.min(jnp.where(logits == m1, eiota, E), axis=1, keepdims=True)
    l2 = jnp.where(eiota == i1, -jnp.inf, logits)
    m2 = jnp.max(l2, axis=1, keepdims=True)
    i2 = jnp.min(jnp.where(l2 == m2, eiota, E), axis=1, keepdims=True)
    sexp = jnp.exp(m2 - m1)
    p1 = 1.0 / (1.0 + sexp)
    p2 = sexp / (1.0 + sexp)
    maskf = ((eiota == i1) | (eiota == i2)).astype(jnp.float32)
    cntf = jnp.sum(maskf, axis=0, keepdims=True)
    cnt_ref[...] = cntf.astype(jnp.int32)
    padded = jnp.ceil(cntf / T) * T
    r8 = jax.lax.broadcasted_iota(jnp.int32, (E, E), 0)
    c8 = jax.lax.broadcasted_iota(jnp.int32, (E, E), 1)
    tri = (r8 < c8).astype(jnp.float32)
    off = jnp.dot(padded, tri, preferred_element_type=jnp.float32)

    def body(b, _):
        r0 = b * 256
        rowi = jax.lax.broadcasted_iota(jnp.int32, (256, S), 0) + r0
        coli = jax.lax.broadcasted_iota(jnp.int32, (256, S), 1)
        lb = (coli < rowi).astype(jnp.float32)
        rank_ref[pl.ds(r0, 256), :] = jnp.dot(
            lb, maskf, preferred_element_type=jnp.float32)
        return 0

    jax.lax.fori_loop(0, S // 256, body, 0)
    posf = off + rank_ref[...]
    sel1 = (eiota == i1).astype(jnp.float32)
    sel2 = (eiota == i2).astype(jnp.float32)
    d1 = jnp.sum(sel1 * posf, axis=1, keepdims=True)
    d2 = jnp.sum(sel2 * posf, axis=1, keepdims=True)
    kiota = jax.lax.broadcasted_iota(jnp.int32, (S, K), 1)
    dest_ref[...] = jnp.where(kiota == 0, d1, d2).astype(jnp.int32)
    w_ref[...] = jnp.where(kiota == 0, p1, p2)


def _router(hid, rw):
    return pl.pallas_call(
        _router_body,
        out_shape=[
            jax.ShapeDtypeStruct((S, K), jnp.int32),
            jax.ShapeDtypeStruct((S, K), jnp.float32),
            jax.ShapeDtypeStruct((1, E), jnp.int32),
        ],
        scratch_shapes=[pltpu.VMEM((S, E), jnp.float32)],
    )(hid, rw)


def _xtile(e, tt, m):
    tiles_e = m[E + e]
    j = m[e] + jnp.maximum(0, jnp.minimum(tt, tiles_e - 1))
    return (jnp.maximum(0, j), 0)


def _ewin(e, tt, m):
    return (e, 0, 0)


def _gateup_body(meta_ref, x_ref, g0, g1, g2, g3, u0, u1, u2, u3, h_ref):
    e = pl.program_id(0)
    tt = pl.program_id(1)

    @pl.when(tt < meta_ref[E + e])
    def _():
        x = x_ref[...].astype(jnp.float32)
        gs = [g0, g1, g2, g3]
        us = [u0, u1, u2, u3]
        g = jnp.zeros((T, FF), jnp.float32)
        u = jnp.zeros((T, FF), jnp.float32)
        for i in range(4):
            xc = x[:, i * HC:(i + 1) * HC]
            g = g + jnp.dot(xc, gs[i][0], preferred_element_type=jnp.float32)
            u = u + jnp.dot(xc, us[i][0], preferred_element_type=jnp.float32)
        g = g * jax.nn.sigmoid(g)
        g = jnp.clip(g, -LIMIT, LIMIT)
        h_ref[...] = (g * u).astype(jnp.bfloat16)


def _down_body(meta_ref, h_ref, d0, d1, d2, d3, w_ref, tok_ref, o_ref):
    e = pl.program_id(0)
    tt = pl.program_id(1)

    @pl.when(jnp.logical_and(e == 0, tt == 0))
    def _():
        o_ref[...] = jnp.zeros((S, H), jnp.float32)

    @pl.when(tt < meta_ref[E + e])
    def _():
        h = h_ref[...].astype(jnp.float32)
        ds = [d0, d1, d2, d3]
        y = jnp.zeros((T, H), jnp.float32)
        for i in range(4):
            y = y + jnp.dot(h[:, i * FC:(i + 1) * FC], ds[i][0],
                            preferred_element_type=jnp.float32)
        yw = y * w_ref[...]
        siota = jax.lax.broadcasted_iota(jnp.int32, (T, S), 1)
        onehot = (siota == tok_ref[...]).astype(jnp.float32)
        scat = jax.lax.dot_general(
            onehot, yw, (((0,), (0,)), ((), ())),
            preferred_element_type=jnp.float32)
        o_ref[...] = o_ref[...] + scat


def _moe(meta, xs, gate_w, up_w, down_w, wsort, tokp):
    gu_spec = pltpu.PrefetchScalarGridSpec(
        num_scalar_prefetch=1,
        grid=(E, MAXTT),
        in_specs=[pl.BlockSpec((T, H), _xtile)] + [
            pl.BlockSpec((1, HC, FF), _ewin) for _ in range(8)
        ],
        out_specs=pl.BlockSpec((T, FF), _xtile),
    )
    g4 = gate_w.reshape(E, 4, HC, FF)
    u4 = up_w.reshape(E, 4, HC, FF)
    gview = [g4[:, i] for i in range(4)] + [u4[:, i] for i in range(4)]
    h = pl.pallas_call(
        _gateup_body,
        grid_spec=gu_spec,
        out_shape=jax.ShapeDtypeStruct((PMAX, FF), jnp.bfloat16),
    )(meta, xs, *gview)

    d4 = down_w.reshape(E, 4, FC, H)
    dview = [d4[:, i] for i in range(4)]
    dn_spec = pltpu.PrefetchScalarGridSpec(
        num_scalar_prefetch=1,
        grid=(E, MAXTT),
        in_specs=[pl.BlockSpec((T, FF), _xtile)] + [
            pl.BlockSpec((1, FC, H), _ewin) for _ in range(4)
        ] + [
            pl.BlockSpec((T, 1), _xtile),
            pl.BlockSpec((T, 1), _xtile),
        ],
        out_specs=pl.BlockSpec((S, H), lambda e, tt, m: (0, 0)),
    )
    return pl.pallas_call(
        _down_body,
        grid_spec=dn_spec,
        out_shape=jax.ShapeDtypeStruct((S, H), jnp.float32),
    )(meta, h, *dview, wsort, tokp)


def kernel(hidden_states, router_weights, gate_w, up_w, down_w):
    hid = hidden_states.reshape(S, H)
    dest, w, cnt = _router(hid, router_weights)
    cnt = cnt.reshape(E)
    tiles_per = ((cnt + (T - 1)) // T).astype(jnp.int32)
    estart = jnp.cumsum(tiles_per) - tiles_per
    meta = jnp.concatenate([estart.astype(jnp.int32), tiles_per])

    destf = dest.reshape(S * K)
    j = jnp.arange(S * K, dtype=jnp.int32)
    tokp = jnp.zeros((PMAX,), jnp.int32).at[destf].set(j >> 1)
    wsort = jnp.zeros((PMAX, 1), jnp.float32).at[destf, 0].set(w.reshape(S * K))
    xs = hid[tokp].astype(jnp.bfloat16)

    out = _moe(meta, xs, gate_w, up_w, down_w, wsort, tokp.reshape(PMAX, 1))
    return out.reshape(1, S, H)


# speedup vs baseline: 1.2010x; 1.2010x over previous
import jax
import jax.numpy as jnp
from jax.experimental import pallas as pl
from jax.experimental.pallas import tpu as pltpu

E = 8
K = 2
H = 1024
FF = 2880
S = 2048
T = 128
NT = 39
PMAX = 5120
LIMIT = 7.0
GROW = 2 * S


def _router_body(hid_ref, rw_ref, dest_ref, w_ref, cnt_ref, rank_ref):
    x = hid_ref[...]
    logits = jnp.dot(x, rw_ref[...], preferred_element_type=jnp.float32)
    eiota = jax.lax.broadcasted_iota(jnp.int32, (S, E), 1)
    m1 = jnp.max(logits, axis=1, keepdims=True)
    i1 = jnp.min(jnp.where(logits == m1, eiota, E), axis=1, keepdims=True)
    l2 = jnp.where(eiota == i1, -jnp.inf, logits)
    m2 = jnp.max(l2, axis=1, keepdims=True)
    i2 = jnp.min(jnp.where(l2 == m2, eiota, E), axis=1, keepdims=True)
    sexp = jnp.exp(m2 - m1)
    p1 = 1.0 / (1.0 + sexp)
    p2 = sexp / (1.0 + sexp)
    maskf = ((eiota == i1) | (eiota == i2)).astype(jnp.float32)
    cntf = jnp.sum(maskf, axis=0, keepdims=True)
    cnt_ref[...] = cntf.astype(jnp.int32)
    padded = jnp.ceil(cntf / T) * T
    r8 = jax.lax.broadcasted_iota(jnp.int32, (E, E), 0)
    c8 = jax.lax.broadcasted_iota(jnp.int32, (E, E), 1)
    tri = (r8 < c8).astype(jnp.float32)
    off = jnp.dot(padded, tri, preferred_element_type=jnp.float32)

    def body(b, _):
        r0 = b * T
        rowi = jax.lax.broadcasted_iota(jnp.int32, (T, S), 0) + r0
        coli = jax.lax.broadcasted_iota(jnp.int32, (T, S), 1)
        lb = (coli < rowi).astype(jnp.float32)
        rank_ref[pl.ds(r0, T), :] = jnp.dot(
            lb, maskf, preferred_element_type=jnp.float32)
        return 0

    jax.lax.fori_loop(0, S // T, body, 0)
    posf = off + rank_ref[...]
    sel1 = (eiota == i1).astype(jnp.float32)
    sel2 = (eiota == i2).astype(jnp.float32)
    d1 = jnp.sum(sel1 * posf, axis=1, keepdims=True)
    d2 = jnp.sum(sel2 * posf, axis=1, keepdims=True)
    kiota = jax.lax.broadcasted_iota(jnp.int32, (S, K), 1)
    dest_ref[...] = jnp.where(kiota == 0, d1, d2).astype(jnp.int32)
    w_ref[...] = jnp.where(kiota == 0, p1, p2)


def _router(hid, rw):
    return pl.pallas_call(
        _router_body,
        out_shape=[
            jax.ShapeDtypeStruct((S, K), jnp.int32),
            jax.ShapeDtypeStruct((S, K), jnp.float32),
            jax.ShapeDtypeStruct((1, E), jnp.int32),
        ],
        scratch_shapes=[pltpu.VMEM((S, E), jnp.float32)],
    )(hid, rw)


MAXTT = S // T


def _xtile(e, tt, m):
    tiles_e = m[E + e]
    j = m[e] + jnp.maximum(0, jnp.minimum(tt, tiles_e - 1))
    return (jnp.maximum(0, j), 0)


def _gateup_body(meta_ref, x_ref, g_ref, u_ref, h_ref):
    e = pl.program_id(0)
    tt = pl.program_id(1)

    @pl.when(tt < meta_ref[E + e])
    def _():
        x = x_ref[...].astype(jnp.float32)
        g = jnp.dot(x, g_ref[0], preferred_element_type=jnp.float32)
        g = g * jax.nn.sigmoid(g)
        g = jnp.clip(g, -LIMIT, LIMIT)
        u = jnp.dot(x, u_ref[0], preferred_element_type=jnp.float32)
        h_ref[...] = (g * u).astype(jnp.bfloat16)


def _down_body(meta_ref, h_ref, d_ref, w_ref, o_ref):
    e = pl.program_id(0)
    tt = pl.program_id(1)

    @pl.when(tt < meta_ref[E + e])
    def _():
        y = jnp.dot(h_ref[...].astype(jnp.float32), d_ref[0],
                    preferred_element_type=jnp.float32)
        o_ref[...] = y * w_ref[...]


def _moe(meta, xs, gate_w, up_w, down_w, wsort):
    gu_spec = pltpu.PrefetchScalarGridSpec(
        num_scalar_prefetch=1,
        grid=(E, MAXTT),
        in_specs=[
            pl.BlockSpec((T, H), _xtile),
            pl.BlockSpec((1, H, FF), lambda e, tt, m: (e, 0, 0)),
            pl.BlockSpec((1, H, FF), lambda e, tt, m: (e, 0, 0)),
        ],
        out_specs=pl.BlockSpec((T, FF), _xtile),
    )
    h = pl.pallas_call(
        _gateup_body,
        grid_spec=gu_spec,
        out_shape=jax.ShapeDtypeStruct((PMAX, FF), jnp.bfloat16),
    )(meta, xs, gate_w, up_w)
    dn_spec = pltpu.PrefetchScalarGridSpec(
        num_scalar_prefetch=1,
        grid=(E, MAXTT),
        in_specs=[
            pl.BlockSpec((T, FF), _xtile),
            pl.BlockSpec((1, FF, H), lambda e, tt, m: (e, 0, 0)),
            pl.BlockSpec((T, 1), _xtile),
        ],
        out_specs=pl.BlockSpec((T, H), _xtile),
    )
    return pl.pallas_call(
        _down_body,
        grid_spec=dn_spec,
        out_shape=jax.ShapeDtypeStruct((PMAX, H), jnp.float32),
    )(meta, h, down_w, wsort)


def kernel(hidden_states, router_weights, gate_w, up_w, down_w):
    hid = hidden_states.reshape(S, H)
    dest, w, cnt = _router(hid, router_weights)
    cnt = cnt.reshape(E)
    tiles_per = ((cnt + (T - 1)) // T).astype(jnp.int32)
    estart = jnp.cumsum(tiles_per) - tiles_per
    meta = jnp.concatenate([estart.astype(jnp.int32), tiles_per])

    destf = dest.reshape(S * K)
    j = jnp.arange(S * K, dtype=jnp.int32)
    payload = (j & 1) * S + (j >> 1)
    destrow = jnp.full((PMAX,), GROW, jnp.int32).at[destf].set(payload)
    wsort = jnp.zeros((PMAX, 1), jnp.float32).at[destf, 0].set(w.reshape(S * K))
    gidx = destrow & (S - 1)
    xs = hid[gidx].astype(jnp.bfloat16)

    y = _moe(meta, xs, gate_w, up_w, down_w, wsort)

    buf = jnp.zeros((2 * S + 8, H), jnp.float32).at[destrow].set(y)
    out = buf[:S] + buf[S:2 * S]
    return out.reshape(1, S, H)
